# baseline (speedup 1.0000x reference)
import jax
import jax.numpy as jnp
from jax.experimental import pallas as pl
from jax.experimental.pallas import tpu as pltpu

_T = 8
_NB = 256
_SEQ = 7
_NC = 14


def _feat_kernel(x_ref, w1_ref, b1_ref, w2_ref, b2_ref, o_ref):
    T = x_ref.shape[0]
    x = x_ref[...].reshape(T, 45, 45)
    xp = jnp.pad(x, ((0, 0), (1, 1), (1, 1)))
    w1 = w1_ref[...]
    b1 = b1_ref[...]
    h = jnp.broadcast_to(b1.reshape(1, 1, 1, 32), (T, 45, 45, 32))
    k = 0
    for dy in range(3):
        for dx in range(3):
            h = h + xp[:, dy:dy + 45, dx:dx + 45, None] * w1[k].reshape(1, 1, 1, 32)
            k += 1
    h = jnp.maximum(h, 0.0)
    h = h[:, :44, :44, :].reshape(T, 22, 2, 22, 2, 32).max(axis=4).max(axis=2)
    hp = jnp.pad(h, ((0, 0), (1, 1), (1, 1), (0, 0)))
    pats = [hp[:, dy:dy + 22, dx:dx + 22, :] for dy in range(3) for dx in range(3)]
    p = jnp.concatenate(pats, axis=-1).reshape(T * 484, 288)
    out = jax.lax.dot_general(p, w2_ref[...], (((1,), (1,)), ((), ())),
                              preferred_element_type=jnp.float32)
    out = jnp.maximum(out + b2_ref[...], 0.0).reshape(T, 22, 22, 64)
    out = out.reshape(T, 11, 2, 11, 2, 64).max(axis=4).max(axis=2)
    o_ref[...] = out.reshape(T, 11 * 11 * 64)


def _fc_kernel(feat_ref, w1_ref, b1_ref, w2_ref, b2_ref, g_ref, len_ref, pos_ref,
               probs_ref, mem_ref, h_scr):
    j = pl.program_id(0)
    nsteps = pl.num_programs(0)
    h = jax.lax.dot_general(feat_ref[...], w1_ref[...], (((1,), (1,)), ((), ())),
                            preferred_element_type=jnp.float32)
    h_scr[:, pl.ds(j * _NB, _NB)] = h

    @pl.when(j == nsteps - 1)
    def _():
        n = feat_ref.shape[0]
        hh = jnp.maximum(h_scr[...] + b1_ref[...], 0.0)
        lg2 = jax.lax.dot_general(hh, w2_ref[...], (((1,), (1,)), ((), ())),
                                  preferred_element_type=jnp.float32) + b2_ref[...]
        m = jnp.max(lg2, axis=-1, keepdims=True)
        e = jnp.exp(lg2 - m)
        sym = e / jnp.sum(e, axis=-1, keepdims=True)
        lg = jnp.log(jnp.clip(sym, 1e-12, 1.0))
        val = lg[:, None, :] + g_ref[...].reshape(n, _SEQ, _NC)
        vmax = jnp.max(val, axis=-1, keepdims=True)
        ii = jax.lax.broadcasted_iota(jnp.int32, (n, _SEQ, _NC), 2)
        idx = jnp.min(jnp.where(val >= vmax, ii, _NC), axis=-1, keepdims=True)
        memb = jnp.max((ii == idx).astype(jnp.float32), axis=1)
        valid = (pos_ref[...] < len_ref[...]).astype(jnp.float32)
        mem_ref[...] = memb
        probs_ref[...] = sym * memb * valid


def kernel(img_seq, conv1_w, conv1_b, conv2_w, conv2_b, fc1_w, fc1_b, fc2_w, fc2_b, img_seq_len):
    B, S = img_seq.shape[0], img_seq.shape[1]
    N = B * S
    x = img_seq.reshape(N, 1, 45, 45).astype(jnp.float32)
    w1r = conv1_w.reshape(32, 9).T
    b1r = conv1_b.reshape(1, 32)
    w2r = conv2_w.transpose(0, 2, 3, 1).reshape(64, 288)
    b2r = conv2_b.reshape(1, 64)
    feat = pl.pallas_call(
        _feat_kernel,
        grid=(N // _T,),
        in_specs=[
            pl.BlockSpec((_T, 1, 45, 45), lambda i: (i, 0, 0, 0)),
            pl.BlockSpec((9, 32), lambda i: (0, 0)),
            pl.BlockSpec((1, 32), lambda i: (0, 0)),
            pl.BlockSpec((64, 288), lambda i: (0, 0)),
            pl.BlockSpec((1, 64), lambda i: (0, 0)),
        ],
        out_specs=pl.BlockSpec((_T, 7744), lambda i: (i, 0)),
        out_shape=jax.ShapeDtypeStruct((N, 7744), jnp.float32),
    )(x, w1r, b1r, w2r, b2r)

    w1p = fc1_w.reshape(1024, 64, 11, 11).transpose(0, 2, 3, 1).reshape(1024, 7744)
    b1p = fc1_b.reshape(1, 1024)
    b2p = fc2_b.reshape(1, _NC)
    g = jax.random.gumbel(jax.random.key(42), (B, S, S, _NC), jnp.float32)
    g = g.reshape(N, S * _NC)
    lens = jnp.repeat(img_seq_len.astype(jnp.int32), S).reshape(N, 1)
    poss = jnp.tile(jnp.arange(S, dtype=jnp.int32), B).reshape(N, 1)

    probs, memb = pl.pallas_call(
        _fc_kernel,
        grid=(1024 // _NB,),
        in_specs=[
            pl.BlockSpec((N, 7744), lambda j: (0, 0)),
            pl.BlockSpec((_NB, 7744), lambda j: (j, 0)),
            pl.BlockSpec((1, 1024), lambda j: (0, 0)),
            pl.BlockSpec((_NC, 1024), lambda j: (0, 0)),
            pl.BlockSpec((1, _NC), lambda j: (0, 0)),
            pl.BlockSpec((N, S * _NC), lambda j: (0, 0)),
            pl.BlockSpec((N, 1), lambda j: (0, 0)),
            pl.BlockSpec((N, 1), lambda j: (0, 0)),
        ],
        out_specs=[
            pl.BlockSpec((N, _NC), lambda j: (0, 0)),
            pl.BlockSpec((N, _NC), lambda j: (0, 0)),
        ],
        out_shape=[
            jax.ShapeDtypeStruct((N, _NC), jnp.float32),
            jax.ShapeDtypeStruct((N, _NC), jnp.float32),
        ],
        scratch_shapes=[pltpu.VMEM((N, 1024), jnp.float32)],
    )(feat, w1p, b1p, fc2_w, b2p, g, lens, poss)

    return probs.reshape(B, S, _NC), memb.reshape(B, S, _NC)
